# baseline (speedup 1.0000x reference)
import jax
import jax.numpy as jnp
from jax import lax
from jax.experimental import pallas as pl
from jax.experimental.pallas import tpu as pltpu
from jax.experimental.pallas import tpu_sc as plsc

_DIM = 96
_D4 = _DIM // 4
_DH = _DIM // 2
_PATCH = 14.0
_NPS = 16
_NP = _NPS * _NPS
_B = 32
_NF = 8192
_L = 16
_ROW = 128
_SAMPLES_PER_STEP = 8
_ROWS = _SAMPLES_PER_STEP * _NP


def _sc_hist(pix_hbm, pers_hbm, h_hbm,
             pix_v, pers_v, h_v, sem):
    c = lax.axis_index("c")
    s = lax.axis_index("s")
    wid = s * 2 + c

    cps = [
        pltpu.async_copy(pix_hbm.at[pl.ds(wid * 2 * _NF, 2 * _NF)], pix_v,
                         sem),
        pltpu.async_copy(pers_hbm.at[pl.ds(wid * 3 * _NF, 3 * _NF)], pers_v,
                         sem),
    ]

    zz = jnp.zeros((_L,), jnp.float32)

    @plsc.parallel_loop(0, _NP * _ROW // _L, unroll=8)
    def _zero(i):
        h_v[pl.ds(i * _L, _L)] = zz

    for cp in cps:
        cp.wait()

    lanes = lax.iota(jnp.int32, _L)
    lanes2 = lanes * 2
    lanes3 = lanes * 3

    @plsc.parallel_loop(0, _NF // _L, unroll=8)
    def _acc(i):
        i2 = i * (2 * _L) + lanes2
        i3 = i * (3 * _L) + lanes3
        x = plsc.load_gather(pix_v, [i2])
        y = plsc.load_gather(pix_v, [i2 + 1])
        a0 = plsc.load_gather(pers_v, [i3])
        a1 = plsc.load_gather(pers_v, [i3 + 1])
        a2 = plsc.load_gather(pers_v, [i3 + 2])
        vf = jnp.sign(x + y + a0 + a1 + a2)
        ix = jnp.minimum(x / _PATCH, float(_NPS - 1)).astype(jnp.int32)
        iy = jnp.minimum(y / _PATCH, float(_NPS - 1)).astype(jnp.int32)
        addr = (iy * _NPS + ix) * _ROW + lanes
        plsc.addupdate_scatter(h_v, [addr], a0)
        plsc.addupdate_scatter(h_v, [addr + _L], a2)
        plsc.addupdate_scatter(h_v, [addr + 2 * _L], vf)

    hsz = _NP * _ROW
    pltpu.sync_copy(h_v, h_hbm.at[pl.ds(wid * hsz, hsz)])


def _tc_dense(h_ref, sp_ref, b1r_ref, b1b_ref, b2w_ref,
              b2b_ref, p1r_ref, p1b_ref, p2w_ref, p2b_ref,
              fws_ref, fwb_ref, fwp_ref, fb_ref, lng_ref, lnb_ref, out_ref):
    hrow = h_ref[...]
    il = lax.broadcasted_iota(jnp.int32, (_ROW, 8), 0)
    iq = lax.broadcasted_iota(jnp.int32, (_ROW, 8), 1)
    sel = ((il // _L) == iq).astype(jnp.float32)
    sums3 = lax.dot_general(hrow, sel, (((1,), (0,)), ((), ())),
                            preferred_element_type=jnp.float32)
    cnt = sums3[:, 2:3]
    mask = cnt > 0.0
    sf = jnp.where(mask, cnt, 1.0)
    pb = jnp.where(mask, sums3[:, 0:1] / sf, 0.0)
    pp = jnp.where(mask, sums3[:, 1:2] / sf, 0.0)

    hb = jnp.maximum(pb * b1r_ref[:] + b1b_ref[:], 0.0)
    hp = jnp.maximum(pp * p1r_ref[:] + p1b_ref[:], 0.0)
    h = jnp.concatenate([hb, hp], axis=-1)

    mb = lax.dot_general(b2w_ref[:], fwb_ref[:], (((0,), (1,)), ((), ())),
                         preferred_element_type=jnp.float32)
    mp = lax.dot_general(p2w_ref[:], fwp_ref[:], (((0,), (1,)), ((), ())),
                         preferred_element_type=jnp.float32)
    m = jnp.concatenate([mb, mp], axis=0)
    cb = lax.dot_general(b2b_ref[:], fwb_ref[:], (((1,), (1,)), ((), ())),
                         preferred_element_type=jnp.float32)
    cp = lax.dot_general(p2b_ref[:], fwp_ref[:], (((1,), (1,)), ((), ())),
                         preferred_element_type=jnp.float32)
    base = (lax.dot_general(sp_ref[:], fws_ref[:], (((1,), (1,)), ((), ())),
                            preferred_element_type=jnp.float32)
            + fb_ref[:] + cb + cp)

    xf = lax.dot_general(h, m, (((1,), (0,)), ((), ())),
                         preferred_element_type=jnp.float32)
    x = xf.reshape(_SAMPLES_PER_STEP, _NP, _DIM) + base[None, :, :]
    mu = jnp.mean(x, axis=-1, keepdims=True)
    d = x - mu
    var = jnp.mean(d * d, axis=-1, keepdims=True)
    xn = d * lax.rsqrt(var + 1e-5)
    out_ref[...] = jnp.tanh(xn * lng_ref[:] + lnb_ref[:])


def kernel(persistence_coords, pixel_coords, spatial_pos, b1_w, b1_b, b2_w,
           b2_b, p1_w, p1_b, p2_w, p2_b, f_w, f_b, ln_g, ln_b, batch_size):
    del batch_size

    pix = pixel_coords.reshape(-1)
    pers = persistence_coords.reshape(-1)

    mesh = plsc.VectorSubcoreMesh(core_axis_name="c", subcore_axis_name="s")
    h_flat = pl.kernel(
        _sc_hist,
        out_type=jax.ShapeDtypeStruct((_B * _NP * _ROW,), jnp.float32),
        mesh=mesh,
        compiler_params=pltpu.CompilerParams(needs_layout_passes=False),
        scratch_types=[
            pltpu.VMEM((2 * _NF,), jnp.float32),
            pltpu.VMEM((3 * _NF,), jnp.float32),
            pltpu.VMEM((_NP * _ROW,), jnp.float32),
            pltpu.SemaphoreType.DMA,
        ],
    )(pix, pers)

    h2 = h_flat.reshape(_B * _NP, _ROW)

    sp = spatial_pos.reshape(_NP, _DH)
    b1r = b1_w.reshape(1, _D4)
    b1br = b1_b.reshape(1, _D4)
    p1r = p1_w.reshape(1, _D4)
    p1br = p1_b.reshape(1, _D4)
    b2br = b2_b.reshape(1, _D4)
    p2br = p2_b.reshape(1, _D4)
    fws = f_w[:, :_DH]
    fwb = f_w[:, _DH:_DH + _D4]
    fwp = f_w[:, _DH + _D4:]
    fbr = f_b.reshape(1, _DIM)
    lngr = ln_g.reshape(1, _DIM)
    lnbr = ln_b.reshape(1, _DIM)

    full = lambda i: (0, 0)
    out = pl.pallas_call(
        _tc_dense,
        grid=(_B // _SAMPLES_PER_STEP,),
        in_specs=[
            pl.BlockSpec((_ROWS, _ROW), lambda i: (i, 0)),
            pl.BlockSpec((_NP, _DH), full),
            pl.BlockSpec((1, _D4), full),
            pl.BlockSpec((1, _D4), full),
            pl.BlockSpec((_D4, _D4), full),
            pl.BlockSpec((1, _D4), full),
            pl.BlockSpec((1, _D4), full),
            pl.BlockSpec((1, _D4), full),
            pl.BlockSpec((_D4, _D4), full),
            pl.BlockSpec((1, _D4), full),
            pl.BlockSpec((_DIM, _DH), full),
            pl.BlockSpec((_DIM, _D4), full),
            pl.BlockSpec((_DIM, _D4), full),
            pl.BlockSpec((1, _DIM), full),
            pl.BlockSpec((1, _DIM), full),
            pl.BlockSpec((1, _DIM), full),
        ],
        out_specs=pl.BlockSpec((_SAMPLES_PER_STEP, _NP, _DIM),
                               lambda i: (i, 0, 0)),
        out_shape=jax.ShapeDtypeStruct((_B, _NP, _DIM), jnp.float32),
    )(h2, sp, b1r, b1br, b2_w, b2br, p1r, p1br, p2_w, p2br,
      fws, fwb, fwp, fbr, lngr, lnbr)
    return out


# speedup vs baseline: 6.7970x; 6.7970x over previous
import jax
import jax.numpy as jnp
from jax import lax
from jax.experimental import pallas as pl
from jax.experimental.pallas import tpu as pltpu
from jax.experimental.pallas import tpu_sc as plsc

_DIM = 96
_D4 = _DIM // 4
_DH = _DIM // 2
_PATCH = 14.0
_NPS = 16
_NP = _NPS * _NPS
_B = 32
_NF = 8192
_L = 16
_ROW = 128
_SAMPLES_PER_STEP = 8
_ROWS = _SAMPLES_PER_STEP * _NP


def _sc_hist(px_hbm, py_hbm, p0_hbm, p1_hbm, p2_hbm, h_hbm,
             px_v, py_v, p0_v, p1_v, p2_v, h_v, sem):
    c = lax.axis_index("c")
    s = lax.axis_index("s")
    wid = s * 2 + c
    base = wid * _NF

    cps = [
        pltpu.async_copy(px_hbm.at[pl.ds(base, _NF)], px_v, sem),
        pltpu.async_copy(py_hbm.at[pl.ds(base, _NF)], py_v, sem),
        pltpu.async_copy(p0_hbm.at[pl.ds(base, _NF)], p0_v, sem),
        pltpu.async_copy(p1_hbm.at[pl.ds(base, _NF)], p1_v, sem),
        pltpu.async_copy(p2_hbm.at[pl.ds(base, _NF)], p2_v, sem),
    ]

    zz = jnp.zeros((_L,), jnp.float32)

    @plsc.parallel_loop(0, _NP * _ROW // _L, unroll=8)
    def _zero(i):
        h_v[pl.ds(i * _L, _L)] = zz

    for cp in cps:
        cp.wait()

    lanes = lax.iota(jnp.int32, _L)

    @plsc.parallel_loop(0, _NF // _L, unroll=8)
    def _acc(i):
        o = i * _L
        x = px_v[pl.ds(o, _L)]
        y = py_v[pl.ds(o, _L)]
        a0 = p0_v[pl.ds(o, _L)]
        a1 = p1_v[pl.ds(o, _L)]
        a2 = p2_v[pl.ds(o, _L)]
        vf = jnp.sign(x + y + a0 + a1 + a2)
        ix = jnp.minimum(x / _PATCH, float(_NPS - 1)).astype(jnp.int32)
        iy = jnp.minimum(y / _PATCH, float(_NPS - 1)).astype(jnp.int32)
        addr = (iy * _NPS + ix) * _ROW + lanes
        plsc.addupdate_scatter(h_v, [addr], a0)
        plsc.addupdate_scatter(h_v, [addr + _L], a2)
        plsc.addupdate_scatter(h_v, [addr + 2 * _L], vf)

    hsz = _NP * _ROW
    pltpu.sync_copy(h_v, h_hbm.at[pl.ds(wid * hsz, hsz)])


def _tc_dense(h_ref, sp_ref, b1r_ref, b1b_ref, b2w_ref,
              b2b_ref, p1r_ref, p1b_ref, p2w_ref, p2b_ref,
              fws_ref, fwb_ref, fwp_ref, fb_ref, lng_ref, lnb_ref, out_ref):
    hrow = h_ref[...]
    il = lax.broadcasted_iota(jnp.int32, (_ROW, 8), 0)
    iq = lax.broadcasted_iota(jnp.int32, (_ROW, 8), 1)
    sel = ((il // _L) == iq).astype(jnp.float32)
    sums3 = lax.dot_general(hrow, sel, (((1,), (0,)), ((), ())),
                            preferred_element_type=jnp.float32)
    cnt = sums3[:, 2:3]
    mask = cnt > 0.0
    sf = jnp.where(mask, cnt, 1.0)
    pb = jnp.where(mask, sums3[:, 0:1] / sf, 0.0)
    pp = jnp.where(mask, sums3[:, 1:2] / sf, 0.0)

    hb = jnp.maximum(pb * b1r_ref[:] + b1b_ref[:], 0.0)
    hp = jnp.maximum(pp * p1r_ref[:] + p1b_ref[:], 0.0)
    h = jnp.concatenate([hb, hp], axis=-1)

    mb = lax.dot_general(b2w_ref[:], fwb_ref[:], (((0,), (1,)), ((), ())),
                         preferred_element_type=jnp.float32)
    mp = lax.dot_general(p2w_ref[:], fwp_ref[:], (((0,), (1,)), ((), ())),
                         preferred_element_type=jnp.float32)
    m = jnp.concatenate([mb, mp], axis=0)
    cb = lax.dot_general(b2b_ref[:], fwb_ref[:], (((1,), (1,)), ((), ())),
                         preferred_element_type=jnp.float32)
    cp = lax.dot_general(p2b_ref[:], fwp_ref[:], (((1,), (1,)), ((), ())),
                         preferred_element_type=jnp.float32)
    base = (lax.dot_general(sp_ref[:], fws_ref[:], (((1,), (1,)), ((), ())),
                            preferred_element_type=jnp.float32)
            + fb_ref[:] + cb + cp)

    xf = lax.dot_general(h, m, (((1,), (0,)), ((), ())),
                         preferred_element_type=jnp.float32)
    x = xf.reshape(_SAMPLES_PER_STEP, _NP, _DIM) + base[None, :, :]
    mu = jnp.mean(x, axis=-1, keepdims=True)
    d = x - mu
    var = jnp.mean(d * d, axis=-1, keepdims=True)
    xn = d * lax.rsqrt(var + 1e-5)
    out_ref[...] = jnp.tanh(xn * lng_ref[:] + lnb_ref[:])


def kernel(persistence_coords, pixel_coords, spatial_pos, b1_w, b1_b, b2_w,
           b2_b, p1_w, p1_b, p2_w, p2_b, f_w, f_b, ln_g, ln_b, batch_size):
    del batch_size

    px = pixel_coords[:, :, 0].reshape(-1)
    py = pixel_coords[:, :, 1].reshape(-1)
    p0 = persistence_coords[:, :, 0].reshape(-1)
    p1 = persistence_coords[:, :, 1].reshape(-1)
    p2 = persistence_coords[:, :, 2].reshape(-1)

    mesh = plsc.VectorSubcoreMesh(core_axis_name="c", subcore_axis_name="s")
    h_flat = pl.kernel(
        _sc_hist,
        out_type=jax.ShapeDtypeStruct((_B * _NP * _ROW,), jnp.float32),
        mesh=mesh,
        compiler_params=pltpu.CompilerParams(needs_layout_passes=False),
        scratch_types=[
            pltpu.VMEM((_NF,), jnp.float32),
            pltpu.VMEM((_NF,), jnp.float32),
            pltpu.VMEM((_NF,), jnp.float32),
            pltpu.VMEM((_NF,), jnp.float32),
            pltpu.VMEM((_NF,), jnp.float32),
            pltpu.VMEM((_NP * _ROW,), jnp.float32),
            pltpu.SemaphoreType.DMA,
        ],
    )(px, py, p0, p1, p2)

    h2 = h_flat.reshape(_B * _NP, _ROW)

    sp = spatial_pos.reshape(_NP, _DH)
    b1r = b1_w.reshape(1, _D4)
    b1br = b1_b.reshape(1, _D4)
    p1r = p1_w.reshape(1, _D4)
    p1br = p1_b.reshape(1, _D4)
    b2br = b2_b.reshape(1, _D4)
    p2br = p2_b.reshape(1, _D4)
    fws = f_w[:, :_DH]
    fwb = f_w[:, _DH:_DH + _D4]
    fwp = f_w[:, _DH + _D4:]
    fbr = f_b.reshape(1, _DIM)
    lngr = ln_g.reshape(1, _DIM)
    lnbr = ln_b.reshape(1, _DIM)

    full = lambda i: (0, 0)
    out = pl.pallas_call(
        _tc_dense,
        grid=(_B // _SAMPLES_PER_STEP,),
        in_specs=[
            pl.BlockSpec((_ROWS, _ROW), lambda i: (i, 0)),
            pl.BlockSpec((_NP, _DH), full),
            pl.BlockSpec((1, _D4), full),
            pl.BlockSpec((1, _D4), full),
            pl.BlockSpec((_D4, _D4), full),
            pl.BlockSpec((1, _D4), full),
            pl.BlockSpec((1, _D4), full),
            pl.BlockSpec((1, _D4), full),
            pl.BlockSpec((_D4, _D4), full),
            pl.BlockSpec((1, _D4), full),
            pl.BlockSpec((_DIM, _DH), full),
            pl.BlockSpec((_DIM, _D4), full),
            pl.BlockSpec((_DIM, _D4), full),
            pl.BlockSpec((1, _DIM), full),
            pl.BlockSpec((1, _DIM), full),
            pl.BlockSpec((1, _DIM), full),
        ],
        out_specs=pl.BlockSpec((_SAMPLES_PER_STEP, _NP, _DIM),
                               lambda i: (i, 0, 0)),
        out_shape=jax.ShapeDtypeStruct((_B, _NP, _DIM), jnp.float32),
    )(h2, sp, b1r, b1br, b2_w, b2br, p1r, p1br, p2_w, p2br,
      fws, fwb, fwp, fbr, lngr, lnbr)
    return out


# speedup vs baseline: 7.0718x; 1.0404x over previous
import jax
import jax.numpy as jnp
from jax import lax
from jax.experimental import pallas as pl
from jax.experimental.pallas import tpu as pltpu
from jax.experimental.pallas import tpu_sc as plsc

_DIM = 96
_D4 = _DIM // 4
_DH = _DIM // 2
_PATCH = 14.0
_NPS = 16
_NP = _NPS * _NPS
_B = 32
_NF = 8192
_L = 16
_ROW = 128
_SAMPLES_PER_STEP = 8
_ROWS = _SAMPLES_PER_STEP * _NP


def _sc_hist(pix_hbm, pers_hbm, h_hbm,
             px_v, py_v, p0_v, p1_v, p2_v, h_v, sem):
    c = lax.axis_index("c")
    s = lax.axis_index("s")
    wid = s * 2 + c
    base = wid * _NF
    bnf = _B * _NF

    cps = [
        pltpu.async_copy(pix_hbm.at[pl.ds(base, _NF)], px_v, sem),
        pltpu.async_copy(pix_hbm.at[pl.ds(bnf + base, _NF)], py_v, sem),
        pltpu.async_copy(pers_hbm.at[pl.ds(base, _NF)], p0_v, sem),
        pltpu.async_copy(pers_hbm.at[pl.ds(bnf + base, _NF)], p1_v, sem),
        pltpu.async_copy(pers_hbm.at[pl.ds(2 * bnf + base, _NF)], p2_v, sem),
    ]

    zz = jnp.zeros((_L,), jnp.float32)

    @plsc.parallel_loop(0, _NP * _ROW // _L, unroll=8)
    def _zero(i):
        h_v[pl.ds(i * _L, _L)] = zz

    for cp in cps:
        cp.wait()

    lanes = lax.iota(jnp.int32, _L)

    @plsc.parallel_loop(0, _NF // _L, unroll=8)
    def _acc(i):
        o = i * _L
        x = px_v[pl.ds(o, _L)]
        y = py_v[pl.ds(o, _L)]
        a0 = p0_v[pl.ds(o, _L)]
        a1 = p1_v[pl.ds(o, _L)]
        a2 = p2_v[pl.ds(o, _L)]
        vf = jnp.sign(x + y + a0 + a1 + a2)
        ix = jnp.minimum(x / _PATCH, float(_NPS - 1)).astype(jnp.int32)
        iy = jnp.minimum(y / _PATCH, float(_NPS - 1)).astype(jnp.int32)
        addr = (iy * _NPS + ix) * _ROW + lanes
        plsc.addupdate_scatter(h_v, [addr], a0)
        plsc.addupdate_scatter(h_v, [addr + _L], a2)
        plsc.addupdate_scatter(h_v, [addr + 2 * _L], vf)

    hsz = _NP * _ROW
    pltpu.sync_copy(h_v, h_hbm.at[pl.ds(wid * hsz, hsz)])


def _tc_dense(h_ref, sp_ref, b1r_ref, b1b_ref, b2w_ref,
              b2b_ref, p1r_ref, p1b_ref, p2w_ref, p2b_ref,
              fws_ref, fwb_ref, fwp_ref, fb_ref, lng_ref, lnb_ref, out_ref):
    hrow = h_ref[...]
    il = lax.broadcasted_iota(jnp.int32, (_ROW, 8), 0)
    iq = lax.broadcasted_iota(jnp.int32, (_ROW, 8), 1)
    sel = ((il // _L) == iq).astype(jnp.float32)
    sums3 = lax.dot_general(hrow, sel, (((1,), (0,)), ((), ())),
                            preferred_element_type=jnp.float32)
    cnt = sums3[:, 2:3]
    mask = cnt > 0.0
    sf = jnp.where(mask, cnt, 1.0)
    pb = jnp.where(mask, sums3[:, 0:1] / sf, 0.0)
    pp = jnp.where(mask, sums3[:, 1:2] / sf, 0.0)

    hb = jnp.maximum(pb * b1r_ref[:] + b1b_ref[:], 0.0)
    hp = jnp.maximum(pp * p1r_ref[:] + p1b_ref[:], 0.0)
    h = jnp.concatenate([hb, hp], axis=-1)

    mb = lax.dot_general(b2w_ref[:], fwb_ref[:], (((0,), (1,)), ((), ())),
                         preferred_element_type=jnp.float32)
    mp = lax.dot_general(p2w_ref[:], fwp_ref[:], (((0,), (1,)), ((), ())),
                         preferred_element_type=jnp.float32)
    m = jnp.concatenate([mb, mp], axis=0)
    cb = lax.dot_general(b2b_ref[:], fwb_ref[:], (((1,), (1,)), ((), ())),
                         preferred_element_type=jnp.float32)
    cp = lax.dot_general(p2b_ref[:], fwp_ref[:], (((1,), (1,)), ((), ())),
                         preferred_element_type=jnp.float32)
    base = (lax.dot_general(sp_ref[:], fws_ref[:], (((1,), (1,)), ((), ())),
                            preferred_element_type=jnp.float32)
            + fb_ref[:] + cb + cp)

    xf = lax.dot_general(h, m, (((1,), (0,)), ((), ())),
                         preferred_element_type=jnp.float32)
    x = xf.reshape(_SAMPLES_PER_STEP, _NP, _DIM) + base[None, :, :]
    mu = jnp.mean(x, axis=-1, keepdims=True)
    d = x - mu
    var = jnp.mean(d * d, axis=-1, keepdims=True)
    xn = d * lax.rsqrt(var + 1e-5)
    out_ref[...] = jnp.tanh(xn * lng_ref[:] + lnb_ref[:])


def kernel(persistence_coords, pixel_coords, spatial_pos, b1_w, b1_b, b2_w,
           b2_b, p1_w, p1_b, p2_w, p2_b, f_w, f_b, ln_g, ln_b, batch_size):
    del batch_size

    pix_t = jnp.moveaxis(pixel_coords, -1, 0).reshape(-1)
    pers_t = jnp.moveaxis(persistence_coords, -1, 0).reshape(-1)

    mesh = plsc.VectorSubcoreMesh(core_axis_name="c", subcore_axis_name="s")
    h_flat = pl.kernel(
        _sc_hist,
        out_type=jax.ShapeDtypeStruct((_B * _NP * _ROW,), jnp.float32),
        mesh=mesh,
        compiler_params=pltpu.CompilerParams(needs_layout_passes=False),
        scratch_types=[
            pltpu.VMEM((_NF,), jnp.float32),
            pltpu.VMEM((_NF,), jnp.float32),
            pltpu.VMEM((_NF,), jnp.float32),
            pltpu.VMEM((_NF,), jnp.float32),
            pltpu.VMEM((_NF,), jnp.float32),
            pltpu.VMEM((_NP * _ROW,), jnp.float32),
            pltpu.SemaphoreType.DMA,
        ],
    )(pix_t, pers_t)

    h2 = h_flat.reshape(_B * _NP, _ROW)

    sp = spatial_pos.reshape(_NP, _DH)
    b1r = b1_w.reshape(1, _D4)
    b1br = b1_b.reshape(1, _D4)
    p1r = p1_w.reshape(1, _D4)
    p1br = p1_b.reshape(1, _D4)
    b2br = b2_b.reshape(1, _D4)
    p2br = p2_b.reshape(1, _D4)
    fws = f_w[:, :_DH]
    fwb = f_w[:, _DH:_DH + _D4]
    fwp = f_w[:, _DH + _D4:]
    fbr = f_b.reshape(1, _DIM)
    lngr = ln_g.reshape(1, _DIM)
    lnbr = ln_b.reshape(1, _DIM)

    full = lambda i: (0, 0)
    out = pl.pallas_call(
        _tc_dense,
        grid=(_B // _SAMPLES_PER_STEP,),
        in_specs=[
            pl.BlockSpec((_ROWS, _ROW), lambda i: (i, 0)),
            pl.BlockSpec((_NP, _DH), full),
            pl.BlockSpec((1, _D4), full),
            pl.BlockSpec((1, _D4), full),
            pl.BlockSpec((_D4, _D4), full),
            pl.BlockSpec((1, _D4), full),
            pl.BlockSpec((1, _D4), full),
            pl.BlockSpec((1, _D4), full),
            pl.BlockSpec((_D4, _D4), full),
            pl.BlockSpec((1, _D4), full),
            pl.BlockSpec((_DIM, _DH), full),
            pl.BlockSpec((_DIM, _D4), full),
            pl.BlockSpec((_DIM, _D4), full),
            pl.BlockSpec((1, _DIM), full),
            pl.BlockSpec((1, _DIM), full),
            pl.BlockSpec((1, _DIM), full),
        ],
        out_specs=pl.BlockSpec((_SAMPLES_PER_STEP, _NP, _DIM),
                               lambda i: (i, 0, 0)),
        out_shape=jax.ShapeDtypeStruct((_B, _NP, _DIM), jnp.float32),
    )(h2, sp, b1r, b1br, b2_w, b2br, p1r, p1br, p2_w, p2br,
      fws, fwb, fwp, fbr, lngr, lnbr)
    return out


# speedup vs baseline: 7.0727x; 1.0001x over previous
import jax
import jax.numpy as jnp
from jax import lax
from jax.experimental import pallas as pl
from jax.experimental.pallas import tpu as pltpu
from jax.experimental.pallas import tpu_sc as plsc

_DIM = 96
_D4 = _DIM // 4
_DH = _DIM // 2
_PATCH = 14.0
_NPS = 16
_NP = _NPS * _NPS
_B = 32
_NF = 8192
_L = 16
_ROW = 128
_SAMPLES_PER_STEP = 8
_ROWS = _SAMPLES_PER_STEP * _NP


def _sc_hist(pix_hbm, pers_hbm, h_hbm,
             px_v, py_v, p0_v, p1_v, p2_v, h_v, sem):
    c = lax.axis_index("c")
    s = lax.axis_index("s")
    wid = s * 2 + c
    base = wid * _NF
    bnf = _B * _NF

    cps = [
        pltpu.async_copy(pix_hbm.at[pl.ds(base, _NF)], px_v, sem),
        pltpu.async_copy(pix_hbm.at[pl.ds(bnf + base, _NF)], py_v, sem),
        pltpu.async_copy(pers_hbm.at[pl.ds(base, _NF)], p0_v, sem),
        pltpu.async_copy(pers_hbm.at[pl.ds(bnf + base, _NF)], p1_v, sem),
        pltpu.async_copy(pers_hbm.at[pl.ds(2 * bnf + base, _NF)], p2_v, sem),
    ]

    zz = jnp.zeros((_L,), jnp.float32)

    @plsc.parallel_loop(0, _NP * _ROW // _L, unroll=8)
    def _zero(i):
        h_v[pl.ds(i * _L, _L)] = zz

    for cp in cps:
        cp.wait()

    lanes = lax.iota(jnp.int32, _L)

    @plsc.parallel_loop(0, _NF // _L, unroll=8)
    def _acc(i):
        o = i * _L
        x = px_v[pl.ds(o, _L)]
        y = py_v[pl.ds(o, _L)]
        a0 = p0_v[pl.ds(o, _L)]
        a1 = p1_v[pl.ds(o, _L)]
        a2 = p2_v[pl.ds(o, _L)]
        vf = jnp.sign(x + y + a0 + a1 + a2)
        ix = jnp.minimum(x / _PATCH, float(_NPS - 1)).astype(jnp.int32)
        iy = jnp.minimum(y / _PATCH, float(_NPS - 1)).astype(jnp.int32)
        addr = (iy * _NPS + ix) * _ROW + lanes
        plsc.addupdate_scatter(h_v, [addr], a0)
        plsc.addupdate_scatter(h_v, [addr + _L], a2)
        plsc.addupdate_scatter(h_v, [addr + 2 * _L], vf)

    hsz = _NP * _ROW
    pltpu.sync_copy(h_v, h_hbm.at[pl.ds(wid * hsz, hsz)])


def _tc_dense(h_ref, sp_ref, b1r_ref, b1b_ref, b2w_ref,
              b2b_ref, p1r_ref, p1b_ref, p2w_ref, p2b_ref,
              fws_ref, fwb_ref, fwp_ref, fb_ref, lng_ref, lnb_ref, out_ref,
              m_sc, base_sc):
    @pl.when(pl.program_id(0) == 0)
    def _fold():
        mb = lax.dot_general(b2w_ref[:], fwb_ref[:], (((0,), (1,)), ((), ())),
                             preferred_element_type=jnp.float32)
        mp = lax.dot_general(p2w_ref[:], fwp_ref[:], (((0,), (1,)), ((), ())),
                             preferred_element_type=jnp.float32)
        m_sc[...] = jnp.concatenate([mb, mp], axis=0)
        cb = lax.dot_general(b2b_ref[:], fwb_ref[:], (((1,), (1,)), ((), ())),
                             preferred_element_type=jnp.float32)
        cp = lax.dot_general(p2b_ref[:], fwp_ref[:], (((1,), (1,)), ((), ())),
                             preferred_element_type=jnp.float32)
        base_sc[...] = (
            lax.dot_general(sp_ref[:], fws_ref[:], (((1,), (1,)), ((), ())),
                            preferred_element_type=jnp.float32)
            + fb_ref[:] + cb + cp)

    hrow = h_ref[...]
    il = lax.broadcasted_iota(jnp.int32, (_ROW, 8), 0)
    iq = lax.broadcasted_iota(jnp.int32, (_ROW, 8), 1)
    sel = ((il // _L) == iq).astype(jnp.float32)
    sums3 = lax.dot_general(hrow, sel, (((1,), (0,)), ((), ())),
                            preferred_element_type=jnp.float32)
    cnt = sums3[:, 2:3]
    mask = cnt > 0.0
    sf = jnp.where(mask, cnt, 1.0)
    pb = jnp.where(mask, sums3[:, 0:1] / sf, 0.0)
    pp = jnp.where(mask, sums3[:, 1:2] / sf, 0.0)

    hb = jnp.maximum(pb * b1r_ref[:] + b1b_ref[:], 0.0)
    hp = jnp.maximum(pp * p1r_ref[:] + p1b_ref[:], 0.0)
    h = jnp.concatenate([hb, hp], axis=-1)

    xf = lax.dot_general(h, m_sc[...], (((1,), (0,)), ((), ())),
                         preferred_element_type=jnp.float32)
    x = xf.reshape(_SAMPLES_PER_STEP, _NP, _DIM) + base_sc[...][None, :, :]
    mu = jnp.mean(x, axis=-1, keepdims=True)
    d = x - mu
    var = jnp.mean(d * d, axis=-1, keepdims=True)
    xn = d * lax.rsqrt(var + 1e-5)
    out_ref[...] = jnp.tanh(xn * lng_ref[:] + lnb_ref[:])


def kernel(persistence_coords, pixel_coords, spatial_pos, b1_w, b1_b, b2_w,
           b2_b, p1_w, p1_b, p2_w, p2_b, f_w, f_b, ln_g, ln_b, batch_size):
    del batch_size

    pix_t = jnp.moveaxis(pixel_coords, -1, 0).reshape(-1)
    pers_t = jnp.moveaxis(persistence_coords, -1, 0).reshape(-1)

    mesh = plsc.VectorSubcoreMesh(core_axis_name="c", subcore_axis_name="s")
    h_flat = pl.kernel(
        _sc_hist,
        out_type=jax.ShapeDtypeStruct((_B * _NP * _ROW,), jnp.float32),
        mesh=mesh,
        compiler_params=pltpu.CompilerParams(needs_layout_passes=False),
        scratch_types=[
            pltpu.VMEM((_NF,), jnp.float32),
            pltpu.VMEM((_NF,), jnp.float32),
            pltpu.VMEM((_NF,), jnp.float32),
            pltpu.VMEM((_NF,), jnp.float32),
            pltpu.VMEM((_NF,), jnp.float32),
            pltpu.VMEM((_NP * _ROW,), jnp.float32),
            pltpu.SemaphoreType.DMA,
        ],
    )(pix_t, pers_t)

    h2 = h_flat.reshape(_B * _NP, _ROW)

    sp = spatial_pos.reshape(_NP, _DH)
    b1r = b1_w.reshape(1, _D4)
    b1br = b1_b.reshape(1, _D4)
    p1r = p1_w.reshape(1, _D4)
    p1br = p1_b.reshape(1, _D4)
    b2br = b2_b.reshape(1, _D4)
    p2br = p2_b.reshape(1, _D4)
    fws = f_w[:, :_DH]
    fwb = f_w[:, _DH:_DH + _D4]
    fwp = f_w[:, _DH + _D4:]
    fbr = f_b.reshape(1, _DIM)
    lngr = ln_g.reshape(1, _DIM)
    lnbr = ln_b.reshape(1, _DIM)

    full = lambda i: (0, 0)
    out = pl.pallas_call(
        _tc_dense,
        grid=(_B // _SAMPLES_PER_STEP,),
        in_specs=[
            pl.BlockSpec((_ROWS, _ROW), lambda i: (i, 0)),
            pl.BlockSpec((_NP, _DH), full),
            pl.BlockSpec((1, _D4), full),
            pl.BlockSpec((1, _D4), full),
            pl.BlockSpec((_D4, _D4), full),
            pl.BlockSpec((1, _D4), full),
            pl.BlockSpec((1, _D4), full),
            pl.BlockSpec((1, _D4), full),
            pl.BlockSpec((_D4, _D4), full),
            pl.BlockSpec((1, _D4), full),
            pl.BlockSpec((_DIM, _DH), full),
            pl.BlockSpec((_DIM, _D4), full),
            pl.BlockSpec((_DIM, _D4), full),
            pl.BlockSpec((1, _DIM), full),
            pl.BlockSpec((1, _DIM), full),
            pl.BlockSpec((1, _DIM), full),
        ],
        out_specs=pl.BlockSpec((_SAMPLES_PER_STEP, _NP, _DIM),
                               lambda i: (i, 0, 0)),
        out_shape=jax.ShapeDtypeStruct((_B, _NP, _DIM), jnp.float32),
        scratch_shapes=[
            pltpu.VMEM((_DH, _DIM), jnp.float32),
            pltpu.VMEM((_NP, _DIM), jnp.float32),
        ],
    )(h2, sp, b1r, b1br, b2_w, b2br, p1r, p1br, p2_w, p2br,
      fws, fwb, fwp, fbr, lngr, lnbr)
    return out


# speedup vs baseline: 7.2383x; 1.0234x over previous
import jax
import jax.numpy as jnp
from jax import lax
from jax.experimental import pallas as pl
from jax.experimental.pallas import tpu as pltpu
from jax.experimental.pallas import tpu_sc as plsc

_DIM = 96
_D4 = _DIM // 4
_DH = _DIM // 2
_PATCH = 14.0
_NPS = 16
_NP = _NPS * _NPS
_B = 32
_NF = 8192
_L = 16
_ROW = 128
_SAMPLES_PER_STEP = 8
_ROWS = _SAMPLES_PER_STEP * _NP


def _sc_hist(pix_hbm, pers_hbm, h_hbm,
             px_v, py_v, p0_v, p1_v, p2_v, h_v, sem):
    c = lax.axis_index("c")
    s = lax.axis_index("s")
    wid = s * 2 + c
    base = wid * _NF
    bnf = _B * _NF

    cps = [
        pltpu.async_copy(pix_hbm.at[pl.ds(base, _NF)], px_v, sem),
        pltpu.async_copy(pix_hbm.at[pl.ds(bnf + base, _NF)], py_v, sem),
        pltpu.async_copy(pers_hbm.at[pl.ds(base, _NF)], p0_v, sem),
        pltpu.async_copy(pers_hbm.at[pl.ds(bnf + base, _NF)], p1_v, sem),
        pltpu.async_copy(pers_hbm.at[pl.ds(2 * bnf + base, _NF)], p2_v, sem),
    ]

    zz = jnp.zeros((_L,), jnp.float32)

    @plsc.parallel_loop(0, _NP, unroll=8)
    def _zero(b):
        o = b * _ROW
        h_v[pl.ds(o, _L)] = zz
        h_v[pl.ds(o + _L, _L)] = zz
        h_v[pl.ds(o + 2 * _L, _L)] = zz

    for cp in cps:
        cp.wait()

    lanes = lax.iota(jnp.int32, _L)

    @plsc.parallel_loop(0, _NF // _L, unroll=8)
    def _acc(i):
        o = i * _L
        x = px_v[pl.ds(o, _L)]
        y = py_v[pl.ds(o, _L)]
        a0 = p0_v[pl.ds(o, _L)]
        a1 = p1_v[pl.ds(o, _L)]
        a2 = p2_v[pl.ds(o, _L)]
        vf = jnp.sign(x + y + a0 + a1 + a2)
        ix = jnp.minimum(x / _PATCH, float(_NPS - 1)).astype(jnp.int32)
        iy = jnp.minimum(y / _PATCH, float(_NPS - 1)).astype(jnp.int32)
        addr = (iy * _NPS + ix) * _ROW + lanes
        plsc.addupdate_scatter(h_v, [addr], a0)
        plsc.addupdate_scatter(h_v, [addr + _L], a2)
        plsc.addupdate_scatter(h_v, [addr + 2 * _L], vf)

    hsz = _NP * _ROW
    pltpu.sync_copy(h_v, h_hbm.at[pl.ds(wid * hsz, hsz)])


def _tc_dense(h_ref, sp_ref, b1r_ref, b1b_ref, b2w_ref,
              b2b_ref, p1r_ref, p1b_ref, p2w_ref, p2b_ref,
              fws_ref, fwb_ref, fwp_ref, fb_ref, lng_ref, lnb_ref, out_ref,
              m_sc, base_sc):
    @pl.when(pl.program_id(0) == 0)
    def _fold():
        mb = lax.dot_general(b2w_ref[:], fwb_ref[:], (((0,), (1,)), ((), ())),
                             preferred_element_type=jnp.float32)
        mp = lax.dot_general(p2w_ref[:], fwp_ref[:], (((0,), (1,)), ((), ())),
                             preferred_element_type=jnp.float32)
        m_sc[...] = jnp.concatenate([mb, mp], axis=0)
        cb = lax.dot_general(b2b_ref[:], fwb_ref[:], (((1,), (1,)), ((), ())),
                             preferred_element_type=jnp.float32)
        cp = lax.dot_general(p2b_ref[:], fwp_ref[:], (((1,), (1,)), ((), ())),
                             preferred_element_type=jnp.float32)
        base_sc[...] = (
            lax.dot_general(sp_ref[:], fws_ref[:], (((1,), (1,)), ((), ())),
                            preferred_element_type=jnp.float32)
            + fb_ref[:] + cb + cp)

    hrow = h_ref[...]
    cnt = jnp.sum(hrow[:, 2 * _L:3 * _L], axis=-1, keepdims=True)
    mask = cnt > 0.0
    sf = jnp.where(mask, cnt, 1.0)
    pb = jnp.where(mask,
                   jnp.sum(hrow[:, :_L], axis=-1, keepdims=True) / sf, 0.0)
    pp = jnp.where(mask,
                   jnp.sum(hrow[:, _L:2 * _L], axis=-1, keepdims=True) / sf,
                   0.0)

    hb = jnp.maximum(pb * b1r_ref[:] + b1b_ref[:], 0.0)
    hp = jnp.maximum(pp * p1r_ref[:] + p1b_ref[:], 0.0)
    h = jnp.concatenate([hb, hp], axis=-1)

    xf = lax.dot_general(h, m_sc[...], (((1,), (0,)), ((), ())),
                         preferred_element_type=jnp.float32)
    x = xf.reshape(_SAMPLES_PER_STEP, _NP, _DIM) + base_sc[...][None, :, :]
    mu = jnp.mean(x, axis=-1, keepdims=True)
    d = x - mu
    var = jnp.mean(d * d, axis=-1, keepdims=True)
    xn = d * lax.rsqrt(var + 1e-5)
    out_ref[...] = jnp.tanh(xn * lng_ref[:] + lnb_ref[:])


def kernel(persistence_coords, pixel_coords, spatial_pos, b1_w, b1_b, b2_w,
           b2_b, p1_w, p1_b, p2_w, p2_b, f_w, f_b, ln_g, ln_b, batch_size):
    del batch_size

    pix_t = jnp.moveaxis(pixel_coords, -1, 0).reshape(-1)
    pers_t = jnp.moveaxis(persistence_coords, -1, 0).reshape(-1)

    mesh = plsc.VectorSubcoreMesh(core_axis_name="c", subcore_axis_name="s")
    h_flat = pl.kernel(
        _sc_hist,
        out_type=jax.ShapeDtypeStruct((_B * _NP * _ROW,), jnp.float32),
        mesh=mesh,
        compiler_params=pltpu.CompilerParams(needs_layout_passes=False),
        scratch_types=[
            pltpu.VMEM((_NF,), jnp.float32),
            pltpu.VMEM((_NF,), jnp.float32),
            pltpu.VMEM((_NF,), jnp.float32),
            pltpu.VMEM((_NF,), jnp.float32),
            pltpu.VMEM((_NF,), jnp.float32),
            pltpu.VMEM((_NP * _ROW,), jnp.float32),
            pltpu.SemaphoreType.DMA,
        ],
    )(pix_t, pers_t)

    h2 = h_flat.reshape(_B * _NP, _ROW)

    sp = spatial_pos.reshape(_NP, _DH)
    b1r = b1_w.reshape(1, _D4)
    b1br = b1_b.reshape(1, _D4)
    p1r = p1_w.reshape(1, _D4)
    p1br = p1_b.reshape(1, _D4)
    b2br = b2_b.reshape(1, _D4)
    p2br = p2_b.reshape(1, _D4)
    fws = f_w[:, :_DH]
    fwb = f_w[:, _DH:_DH + _D4]
    fwp = f_w[:, _DH + _D4:]
    fbr = f_b.reshape(1, _DIM)
    lngr = ln_g.reshape(1, _DIM)
    lnbr = ln_b.reshape(1, _DIM)

    full = lambda i: (0, 0)
    out = pl.pallas_call(
        _tc_dense,
        grid=(_B // _SAMPLES_PER_STEP,),
        in_specs=[
            pl.BlockSpec((_ROWS, _ROW), lambda i: (i, 0)),
            pl.BlockSpec((_NP, _DH), full),
            pl.BlockSpec((1, _D4), full),
            pl.BlockSpec((1, _D4), full),
            pl.BlockSpec((_D4, _D4), full),
            pl.BlockSpec((1, _D4), full),
            pl.BlockSpec((1, _D4), full),
            pl.BlockSpec((1, _D4), full),
            pl.BlockSpec((_D4, _D4), full),
            pl.BlockSpec((1, _D4), full),
            pl.BlockSpec((_DIM, _DH), full),
            pl.BlockSpec((_DIM, _D4), full),
            pl.BlockSpec((_DIM, _D4), full),
            pl.BlockSpec((1, _DIM), full),
            pl.BlockSpec((1, _DIM), full),
            pl.BlockSpec((1, _DIM), full),
        ],
        out_specs=pl.BlockSpec((_SAMPLES_PER_STEP, _NP, _DIM),
                               lambda i: (i, 0, 0)),
        out_shape=jax.ShapeDtypeStruct((_B, _NP, _DIM), jnp.float32),
        scratch_shapes=[
            pltpu.VMEM((_DH, _DIM), jnp.float32),
            pltpu.VMEM((_NP, _DIM), jnp.float32),
        ],
    )(h2, sp, b1r, b1br, b2_w, b2br, p1r, p1br, p2_w, p2br,
      fws, fwb, fwp, fbr, lngr, lnbr)
    return out
